# baseline (speedup 1.0000x reference)
import functools

import jax
import jax.numpy as jnp
from jax import lax
from jax.experimental import pallas as pl
from jax.experimental.pallas import tpu as pltpu
from jax.experimental.pallas import tpu_sc as plsc

MIN_NORM = 1e-15
EPS = 1e-5

N_NODES = 10000
D = 128
N_EDGES = 320000
NFC = 16
FPC = D // NFC
NPR = FPC // 2
NC = 2
EPR = N_EDGES // NC
CHUNK = 2000
NG = CHUNK // 16
NCHUNKS = EPR // CHUNK
TBL = N_NODES * NPR
OSTR = 327680


def _tc_norms(z):
    RB = 2000

    def body(z_ref, o_ref):
        x = z_ref[...]
        o_ref[...] = jnp.sum(x * x, axis=1, keepdims=True)

    return pl.pallas_call(
        body,
        grid=(N_NODES // RB,),
        in_specs=[pl.BlockSpec((RB, D), lambda i: (i, 0))],
        out_specs=pl.BlockSpec((RB, 1), lambda i: (i, 0)),
        out_shape=jax.ShapeDtypeStruct((N_NODES, 1), jnp.float32),
    )(z)


def _sc_dots(zt, norms, ei):
    mesh = plsc.VectorSubcoreMesh(core_axis_name="c", subcore_axis_name="s")

    @functools.partial(
        pl.kernel,
        mesh=mesh,
        compiler_params=pltpu.CompilerParams(needs_layout_passes=False),
        out_type=[
            jax.ShapeDtypeStruct((NFC * OSTR,), jnp.float32),
            jax.ShapeDtypeStruct((OSTR,), jnp.float32),
            jax.ShapeDtypeStruct((OSTR,), jnp.float32),
        ],
        scratch_types=[
            pltpu.VMEM((TBL,), jnp.int32),
            pltpu.VMEM((N_NODES,), jnp.float32),
            pltpu.VMEM((CHUNK,), jnp.int32),
            pltpu.VMEM((CHUNK,), jnp.int32),
            pltpu.VMEM((CHUNK,), jnp.int32),
            pltpu.VMEM((CHUNK,), jnp.int32),
            pltpu.VMEM((CHUNK,), jnp.float32),
            pltpu.VMEM((CHUNK,), jnp.float32),
            pltpu.VMEM((CHUNK,), jnp.float32),
            pltpu.VMEM((CHUNK,), jnp.float32),
            pltpu.SemaphoreType.DMA,
            pltpu.SemaphoreType.DMA,
            pltpu.SemaphoreType.DMA,
            pltpu.SemaphoreType.DMA,
            pltpu.SemaphoreType.DMA,
        ],
    )
    def k(zt_hbm, norms_hbm, ei_hbm, s_hbm, n1_hbm, n2_hbm,
          tbl, ntbl, sid0, sid1, did0, did1, sb0, sb1, n1b, n2b,
          sin0, sin1, sout0, sout1, snrm):
        core = lax.axis_index("c")
        fc = lax.axis_index("s")
        pltpu.sync_copy(zt_hbm.at[pl.ds(fc * TBL, TBL)], tbl)
        pltpu.sync_copy(norms_hbm, ntbl)
        base0 = core * EPR
        sids = (sid0, sid1)
        dids = (did0, did1)
        sbs = (sb0, sb1)
        sins = (sin0, sin1)
        souts = (sout0, sout1)

        def in_copies(i, p):
            base = base0 + i * CHUNK
            return (
                pltpu.make_async_copy(
                    ei_hbm.at[pl.ds(base, CHUNK)], sids[p], sins[p]),
                pltpu.make_async_copy(
                    ei_hbm.at[pl.ds(N_EDGES + base, CHUNK)], dids[p], sins[p]),
            )

        def out_copy(i, p):
            obase = fc * OSTR + base0 + i * CHUNK
            return pltpu.make_async_copy(
                sbs[p], s_hbm.at[pl.ds(obase, CHUNK)], souts[p])

        def nrm_copies(i):
            base = base0 + i * CHUNK
            return (
                pltpu.make_async_copy(
                    n1b, n1_hbm.at[pl.ds(base, CHUNK)], snrm),
                pltpu.make_async_copy(
                    n2b, n2_hbm.at[pl.ds(base, CHUNK)], snrm),
            )

        for c in in_copies(0, 0):
            c.start()

        def chunk_body(i, p, has_next):
            base = base0 + i * CHUNK
            sid, did, sb = sids[p], dids[p], sbs[p]

            @pl.when(has_next)
            def _():
                for c in in_copies(i + 1, 1 - p):
                    c.start()

            for c in in_copies(i, p):
                c.wait()

            @pl.when(i >= 2)
            def _():
                out_copy(i - 2, p).wait()

            def grp(g, carry):
                off = g * 16
                si = sid[pl.ds(off, 16)]
                di = did[pl.ds(off, 16)]
                ss = jnp.zeros((16,), jnp.float32)
                for prow in range(NPR):
                    pv = jnp.full((16,), prow * N_NODES, jnp.int32)
                    wa = plsc.load_gather(tbl, [si + pv])
                    wb = plsc.load_gather(tbl, [di + pv])
                    a0, a1 = plsc.unpack(
                        plsc.bitcast(wa, jnp.bfloat16),
                        format=plsc.PackFormat.INTERLEAVED)
                    b0, b1 = plsc.unpack(
                        plsc.bitcast(wb, jnp.bfloat16),
                        format=plsc.PackFormat.INTERLEAVED)
                    ss = ss + (a0 * b0 + a1 * b1)
                sb[pl.ds(off, 16)] = ss
                return carry

            lax.fori_loop(0, NG, grp, 0, unroll=25)
            out_copy(i, p).start()

            @pl.when(lax.rem(i, NFC) == fc)
            def _():
                @pl.when(i >= NFC)
                def _():
                    for c in nrm_copies(i - NFC):
                        c.wait()

                def ngrp(g, carry):
                    off = g * 16
                    si = sid[pl.ds(off, 16)]
                    di = did[pl.ds(off, 16)]
                    n1b[pl.ds(off, 16)] = plsc.load_gather(ntbl, [si])
                    n2b[pl.ds(off, 16)] = plsc.load_gather(ntbl, [di])
                    return carry

                lax.fori_loop(0, NG, ngrp, 0, unroll=5)
                for c in nrm_copies(i):
                    c.start()

        def pair(j, carry):
            i0 = j * 2
            chunk_body(i0, 0, jnp.bool_(True))
            chunk_body(i0 + 1, 1, i0 + 2 < NCHUNKS)
            return carry

        lax.fori_loop(0, NCHUNKS // 2, pair, 0)
        out_copy(NCHUNKS - 2, 0).wait()
        out_copy(NCHUNKS - 1, 1).wait()
        for c in nrm_copies(NCHUNKS - NFC + fc):
            c.wait()

    return k(zt, norms, ei)


def _tc_finish(sp, n1, n2):
    EB = 16384
    NB = OSTR // EB

    def body(*refs):
        s_refs = refs[:NFC]
        n1_ref, n2_ref, o_ref = refs[NFC], refs[NFC + 1], refs[NFC + 2]
        x2 = n1_ref[...]
        y2 = n2_ref[...]
        s = s_refs[0][...]
        for r in s_refs[1:]:
            s = s + r[...]
        a = 1.0 - 2.0 * s + y2
        b = 1.0 - x2
        numsq = a * a * x2 - 2.0 * a * b * s + b * b * y2
        den = jnp.maximum(1.0 - 2.0 * s + x2 * y2, MIN_NORM)
        r = numsq / (den * den)
        norm = jnp.sqrt(jnp.maximum(r, MIN_NORM))
        t = jnp.clip(norm, -1.0 + EPS, 1.0 - EPS)
        dist = jnp.log1p(t) - jnp.log1p(-t)
        o_ref[...] = -(dist * dist)

    in_specs = [
        pl.BlockSpec((EB,), functools.partial(lambda fc, i: (fc * NB + i,), fc))
        for fc in range(NFC)
    ]
    in_specs += [pl.BlockSpec((EB,), lambda i: (i,))] * 2
    out = pl.pallas_call(
        body,
        grid=(NB,),
        in_specs=in_specs,
        out_specs=pl.BlockSpec((EB,), lambda i: (i,)),
        out_shape=jax.ShapeDtypeStruct((OSTR,), jnp.float32),
    )(*([sp] * NFC), n1, n2)
    return out[:N_EDGES]


def kernel(z, edge_index):
    zb = z.astype(jnp.bfloat16).T
    zp = zb.reshape(D // 2, 2, N_NODES).transpose(0, 2, 1)
    zt = lax.bitcast_convert_type(zp, jnp.int32).reshape(-1)
    norms = _tc_norms(z)
    sp, n1, n2 = _sc_dots(zt, norms.reshape(-1), edge_index.reshape(-1))
    return _tc_finish(sp, n1, n2)


# speedup vs baseline: 1.0893x; 1.0893x over previous
import functools

import jax
import jax.numpy as jnp
from jax import lax
from jax.experimental import pallas as pl
from jax.experimental.pallas import tpu as pltpu
from jax.experimental.pallas import tpu_sc as plsc

MIN_NORM = 1e-15
EPS = 1e-5

N_NODES = 10000
D = 128
N_EDGES = 320000
NFC = 16
FPC = D // NFC
NPR = FPC // 2
NC = 2
EPR = N_EDGES // NC
CHUNK = 2000
NG = CHUNK // 16
NCHUNKS = EPR // CHUNK
TBL = N_NODES * NPR
OSTR = 327680


def _tc_norms(z):
    RB = 2000

    def body(z_ref, o_ref, w_ref):
        x = z_ref[...]
        o_ref[...] = jnp.sum(x * x, axis=1, keepdims=True)
        f_ids = lax.broadcasted_iota(jnp.int32, (D, D // 2), 0)
        q_ids = lax.broadcasted_iota(jnp.int32, (D, D // 2), 1)
        s_even = (f_ids == 2 * q_ids).astype(jnp.float32)
        s_odd = (f_ids == 2 * q_ids + 1).astype(jnp.float32)
        lo = jnp.dot(x, s_even).astype(jnp.bfloat16)
        hi = jnp.dot(x, s_odd).astype(jnp.bfloat16)
        ulo = lax.bitcast_convert_type(lo, jnp.uint16).astype(jnp.uint32)
        uhi = lax.bitcast_convert_type(hi, jnp.uint16).astype(jnp.uint32)
        w_ref[...] = (ulo | (uhi << 16)).astype(jnp.int32)

    return pl.pallas_call(
        body,
        grid=(N_NODES // RB,),
        in_specs=[pl.BlockSpec((RB, D), lambda i: (i, 0))],
        out_specs=[
            pl.BlockSpec((RB, 1), lambda i: (i, 0)),
            pl.BlockSpec((RB, D // 2), lambda i: (i, 0)),
        ],
        out_shape=[
            jax.ShapeDtypeStruct((N_NODES, 1), jnp.float32),
            jax.ShapeDtypeStruct((N_NODES, D // 2), jnp.int32),
        ],
    )(z)


def _sc_dots(zt, norms, ei):
    mesh = plsc.VectorSubcoreMesh(core_axis_name="c", subcore_axis_name="s")

    @functools.partial(
        pl.kernel,
        mesh=mesh,
        compiler_params=pltpu.CompilerParams(needs_layout_passes=False),
        out_type=[
            jax.ShapeDtypeStruct((NFC * OSTR,), jnp.float32),
            jax.ShapeDtypeStruct((OSTR,), jnp.float32),
            jax.ShapeDtypeStruct((OSTR,), jnp.float32),
        ],
        scratch_types=[
            pltpu.VMEM((TBL,), jnp.int32),
            pltpu.VMEM((N_NODES,), jnp.float32),
            pltpu.VMEM((CHUNK,), jnp.int32),
            pltpu.VMEM((CHUNK,), jnp.int32),
            pltpu.VMEM((CHUNK,), jnp.int32),
            pltpu.VMEM((CHUNK,), jnp.int32),
            pltpu.VMEM((CHUNK,), jnp.float32),
            pltpu.VMEM((CHUNK,), jnp.float32),
            pltpu.VMEM((CHUNK,), jnp.float32),
            pltpu.VMEM((CHUNK,), jnp.float32),
            pltpu.SemaphoreType.DMA,
            pltpu.SemaphoreType.DMA,
            pltpu.SemaphoreType.DMA,
            pltpu.SemaphoreType.DMA,
            pltpu.SemaphoreType.DMA,
        ],
    )
    def k(zt_hbm, norms_hbm, ei_hbm, s_hbm, n1_hbm, n2_hbm,
          tbl, ntbl, sid0, sid1, did0, did1, sb0, sb1, n1b, n2b,
          sin0, sin1, sout0, sout1, snrm):
        core = lax.axis_index("c")
        fc = lax.axis_index("s")
        pltpu.sync_copy(zt_hbm.at[pl.ds(fc * TBL, TBL)], tbl)
        pltpu.sync_copy(norms_hbm, ntbl)
        base0 = core * EPR
        sids = (sid0, sid1)
        dids = (did0, did1)
        sbs = (sb0, sb1)
        sins = (sin0, sin1)
        souts = (sout0, sout1)

        def in_copies(i, p):
            base = base0 + i * CHUNK
            return (
                pltpu.make_async_copy(
                    ei_hbm.at[pl.ds(base, CHUNK)], sids[p], sins[p]),
                pltpu.make_async_copy(
                    ei_hbm.at[pl.ds(N_EDGES + base, CHUNK)], dids[p], sins[p]),
            )

        def out_copy(i, p):
            obase = fc * OSTR + base0 + i * CHUNK
            return pltpu.make_async_copy(
                sbs[p], s_hbm.at[pl.ds(obase, CHUNK)], souts[p])

        def nrm_copies(i):
            base = base0 + i * CHUNK
            return (
                pltpu.make_async_copy(
                    n1b, n1_hbm.at[pl.ds(base, CHUNK)], snrm),
                pltpu.make_async_copy(
                    n2b, n2_hbm.at[pl.ds(base, CHUNK)], snrm),
            )

        for c in in_copies(0, 0):
            c.start()

        def chunk_body(i, p, has_next):
            base = base0 + i * CHUNK
            sid, did, sb = sids[p], dids[p], sbs[p]

            @pl.when(has_next)
            def _():
                for c in in_copies(i + 1, 1 - p):
                    c.start()

            for c in in_copies(i, p):
                c.wait()

            @pl.when(i >= 2)
            def _():
                out_copy(i - 2, p).wait()

            def grp(g, carry):
                off = g * 16
                si = sid[pl.ds(off, 16)]
                di = did[pl.ds(off, 16)]
                ss = jnp.zeros((16,), jnp.float32)
                for prow in range(NPR):
                    pv = jnp.full((16,), prow * N_NODES, jnp.int32)
                    wa = plsc.load_gather(tbl, [si + pv])
                    wb = plsc.load_gather(tbl, [di + pv])
                    a0, a1 = plsc.unpack(
                        plsc.bitcast(wa, jnp.bfloat16),
                        format=plsc.PackFormat.INTERLEAVED)
                    b0, b1 = plsc.unpack(
                        plsc.bitcast(wb, jnp.bfloat16),
                        format=plsc.PackFormat.INTERLEAVED)
                    ss = ss + (a0 * b0 + a1 * b1)
                sb[pl.ds(off, 16)] = ss
                return carry

            lax.fori_loop(0, NG, grp, 0, unroll=5)
            out_copy(i, p).start()

            @pl.when(lax.rem(i, NFC) == fc)
            def _():
                @pl.when(i >= NFC)
                def _():
                    for c in nrm_copies(i - NFC):
                        c.wait()

                def ngrp(g, carry):
                    off = g * 16
                    si = sid[pl.ds(off, 16)]
                    di = did[pl.ds(off, 16)]
                    n1b[pl.ds(off, 16)] = plsc.load_gather(ntbl, [si])
                    n2b[pl.ds(off, 16)] = plsc.load_gather(ntbl, [di])
                    return carry

                lax.fori_loop(0, NG, ngrp, 0, unroll=5)
                for c in nrm_copies(i):
                    c.start()

        def pair(j, carry):
            i0 = j * 2
            chunk_body(i0, 0, jnp.bool_(True))
            chunk_body(i0 + 1, 1, i0 + 2 < NCHUNKS)
            return carry

        lax.fori_loop(0, NCHUNKS // 2, pair, 0)
        out_copy(NCHUNKS - 2, 0).wait()
        out_copy(NCHUNKS - 1, 1).wait()
        for c in nrm_copies(NCHUNKS - NFC + fc):
            c.wait()

    return k(zt, norms, ei)


def _tc_finish(sp, n1, n2):
    EB = 16384
    NB = OSTR // EB

    def body(*refs):
        s_refs = refs[:NFC]
        n1_ref, n2_ref, o_ref = refs[NFC], refs[NFC + 1], refs[NFC + 2]
        x2 = n1_ref[...]
        y2 = n2_ref[...]
        s = s_refs[0][...]
        for r in s_refs[1:]:
            s = s + r[...]
        a = 1.0 - 2.0 * s + y2
        b = 1.0 - x2
        numsq = a * a * x2 - 2.0 * a * b * s + b * b * y2
        den = jnp.maximum(1.0 - 2.0 * s + x2 * y2, MIN_NORM)
        r = numsq / (den * den)
        norm = jnp.sqrt(jnp.maximum(r, MIN_NORM))
        t = jnp.clip(norm, -1.0 + EPS, 1.0 - EPS)
        dist = jnp.log1p(t) - jnp.log1p(-t)
        o_ref[...] = -(dist * dist)

    in_specs = [
        pl.BlockSpec((EB,), functools.partial(lambda fc, i: (fc * NB + i,), fc))
        for fc in range(NFC)
    ]
    in_specs += [pl.BlockSpec((EB,), lambda i: (i,))] * 2
    out = pl.pallas_call(
        body,
        grid=(NB,),
        in_specs=in_specs,
        out_specs=pl.BlockSpec((EB,), lambda i: (i,)),
        out_shape=jax.ShapeDtypeStruct((OSTR,), jnp.float32),
    )(*([sp] * NFC), n1, n2)
    return out[:N_EDGES]


def kernel(z, edge_index):
    norms, w = _tc_norms(z)
    zt = w.T.reshape(-1)
    sp, n1, n2 = _sc_dots(zt, norms.reshape(-1), edge_index.reshape(-1))
    return _tc_finish(sp, n1, n2)
